# baseline (speedup 1.0000x reference)
import jax
import jax.numpy as jnp
from jax.experimental import pallas as pl
from jax.experimental.pallas import tpu as pltpu

_B = 16384
_D = 256
_H = 256
_L = 32
_K = 512
_BETA = 0.25
_BT = 4096


def _fused_body(x_ref, eW1_ref, eb1_ref, eW2_ref, eb2_ref, eW3_ref, eb3_ref,
                dW1_ref, db1_ref, dW2_ref, db2_ref, dW3_ref, db3_ref,
                c2_ref,
                cb0h1_ref, cb1h1_ref, cb2h1_ref,
                cb0c_ref, cb1c_ref, cb2c_ref,
                recon_ref, codes_ref, loss_ref):
    f32 = jnp.float32
    bf16 = jnp.bfloat16
    z = jnp.maximum(
        jnp.dot(x_ref[...].astype(bf16), eW1_ref[...],
                preferred_element_type=f32)
        + eb1_ref[...], 0.0)
    z = jnp.maximum(
        jnp.dot(z.astype(bf16), eW2_ref[...], preferred_element_type=f32)
        + eb2_ref[...], 0.0)
    z = (jnp.dot(z.astype(bf16), eW3_ref[...], preferred_element_type=f32)
         + eb3_ref[...])

    r = z
    quant = jnp.zeros_like(z)
    loss_acc = jnp.float32(0.0)
    levels = (
        (cb0h1_ref, cb0c_ref),
        (cb1h1_ref, cb1c_ref),
        (cb2h1_ref, cb2c_ref),
    )
    for lvl, (h1_ref, cbc_ref) in enumerate(levels):
        r2 = jnp.sum(r * r, axis=1, keepdims=True)
        cross = jax.lax.dot_general(
            r.astype(bf16), h1_ref[...], (((1,), (1,)), ((), ())),
            preferred_element_type=f32)
        c2 = c2_ref[lvl:lvl + 1, :]
        d2 = r2 - 2.0 * cross + c2
        mind = jnp.min(d2, axis=1, keepdims=True)
        iota_f = jax.lax.broadcasted_iota(jnp.int32, d2.shape, 1).astype(f32)
        code_f = jnp.min(jnp.where(d2 == mind, iota_f, jnp.float32(_K)),
                         axis=1, keepdims=True)
        codes_ref[:, lvl:lvl + 1] = code_f.astype(jnp.int32)
        onehot = (iota_f == code_f).astype(bf16)
        s = jnp.dot(onehot, cbc_ref[...], preferred_element_type=f32)
        e = (s[:, 128:128 + _L] + s[:, 256:256 + _L]) + s[:, :_L]
        quant = quant + r + (e - r)
        r = r - e
        loss_acc = loss_acc + jnp.sum(r * r)

    h = jnp.maximum(
        jnp.dot(quant.astype(bf16), dW1_ref[...], preferred_element_type=f32)
        + db1_ref[...], 0.0)
    h = jnp.maximum(
        jnp.dot(h.astype(bf16), dW2_ref[...], preferred_element_type=f32)
        + db2_ref[...], 0.0)
    recon_ref[...] = (
        jnp.dot(h.astype(bf16), dW3_ref[...], preferred_element_type=f32)
        + db3_ref[...])

    loss_ref[...] = loss_acc.reshape(1, 1, 1)


def _split3(cb):
    bf16 = jnp.bfloat16
    f32 = jnp.float32
    h1 = cb.astype(bf16)
    d1 = cb - h1.astype(f32)
    h2 = d1.astype(bf16)
    d2 = d1 - h2.astype(f32)
    h3 = d2.astype(bf16)
    pad = jnp.zeros((cb.shape[0], 128 - cb.shape[1]), dtype=bf16)
    return h1, jnp.concatenate([h1, pad, h2, pad, h3, pad], axis=1)


@jax.jit
def kernel(x, eW1, eb1, eW2, eb2, eW3, eb3, dW1, db1, dW2, db2, dW3, db3,
           cb0, cb1, cb2):
    grid = _B // _BT
    bf16 = jnp.bfloat16
    rep = lambda i: (0, 0)

    c2all = jnp.stack([jnp.sum(cb * cb, axis=1) for cb in (cb0, cb1, cb2)])
    his, cbcats = zip(*[_split3(cb) for cb in (cb0, cb1, cb2)])

    recon, codes, loss_parts = pl.pallas_call(
        _fused_body,
        grid=(grid,),
        in_specs=[
            pl.BlockSpec((_BT, _D), lambda i: (i, 0)),
            pl.BlockSpec((_D, _H), rep),
            pl.BlockSpec((1, _H), rep),
            pl.BlockSpec((_H, _H), rep),
            pl.BlockSpec((1, _H), rep),
            pl.BlockSpec((_H, _L), rep),
            pl.BlockSpec((1, _L), rep),
            pl.BlockSpec((_L, _H), rep),
            pl.BlockSpec((1, _H), rep),
            pl.BlockSpec((_H, _H), rep),
            pl.BlockSpec((1, _H), rep),
            pl.BlockSpec((_H, _D), rep),
            pl.BlockSpec((1, _D), rep),
            pl.BlockSpec((3, _K), rep),
        ] + [pl.BlockSpec((_K, _L), rep)] * 3 \
          + [pl.BlockSpec((_K, 384), rep)] * 3,
        out_specs=[
            pl.BlockSpec((_BT, _D), lambda i: (i, 0)),
            pl.BlockSpec((_BT, 3), lambda i: (i, 0)),
            pl.BlockSpec((1, 1, 1), lambda i: (i, 0, 0)),
        ],
        out_shape=[
            jax.ShapeDtypeStruct((_B, _D), jnp.float32),
            jax.ShapeDtypeStruct((_B, 3), jnp.int32),
            jax.ShapeDtypeStruct((grid, 1, 1), jnp.float32),
        ],
        compiler_params=pltpu.CompilerParams(
            dimension_semantics=("parallel",)),
    )(x, eW1.astype(bf16), eb1.reshape(1, -1),
      eW2.astype(bf16), eb2.reshape(1, -1), eW3.astype(bf16),
      eb3.reshape(1, -1), dW1.astype(bf16), db1.reshape(1, -1),
      dW2.astype(bf16), db2.reshape(1, -1), dW3.astype(bf16),
      db3.reshape(1, -1), c2all, *his, *cbcats)

    loss = jnp.sum(loss_parts) * ((1.0 + _BETA) / (_B * _L))
    return recon, codes, loss


# speedup vs baseline: 1.1237x; 1.1237x over previous
import jax
import jax.numpy as jnp
from jax.experimental import pallas as pl
from jax.experimental.pallas import tpu as pltpu

_B = 16384
_D = 256
_H = 256
_L = 32
_K = 512
_BETA = 0.25
_BT = 2048


def _fused_body(x_ref, eW1_ref, eb1_ref, eW2_ref, eb2_ref, eW3_ref, eb3_ref,
                dW1_ref, db1_ref, dW2_ref, db2_ref, dW3_ref, db3_ref,
                c2_ref, cb0_ref, cb1_ref, cb2_ref,
                recon_ref, codes_ref, loss_ref):
    f32 = jnp.float32
    bf16 = jnp.bfloat16
    z = jnp.maximum(
        jnp.dot(x_ref[...].astype(bf16), eW1_ref[...].astype(bf16),
                preferred_element_type=f32)
        + eb1_ref[...], 0.0)
    z = jnp.maximum(
        jnp.dot(z.astype(bf16), eW2_ref[...].astype(bf16),
                preferred_element_type=f32)
        + eb2_ref[...], 0.0)
    z = (jnp.dot(z.astype(bf16), eW3_ref[...].astype(bf16),
                 preferred_element_type=f32)
         + eb3_ref[...])

    r = z
    quant = jnp.zeros_like(z)
    loss_acc = jnp.float32(0.0)
    zpad = jnp.zeros((_K, 128 - _L), dtype=bf16)
    for lvl, cb_ref in enumerate((cb0_ref, cb1_ref, cb2_ref)):
        cb = cb_ref[...]
        h1 = cb.astype(bf16)
        d1 = cb - h1.astype(f32)
        h2 = d1.astype(bf16)
        h3 = (d1 - h2.astype(f32)).astype(bf16)
        hcat = jnp.concatenate([h1, zpad, h2, zpad, h3, zpad], axis=1)

        r2 = jnp.sum(r * r, axis=1, keepdims=True)
        cross2 = jax.lax.dot_general(
            r.astype(bf16), h1 * bf16(2.0), (((1,), (1,)), ((), ())),
            preferred_element_type=f32)
        c2 = c2_ref[lvl:lvl + 1, :]
        d2 = r2 - cross2 + c2
        mind = jnp.min(d2, axis=1, keepdims=True)
        iota_f = jax.lax.broadcasted_iota(jnp.int32, d2.shape, 1).astype(f32)
        code_f = jnp.min(jnp.where(d2 == mind, iota_f, jnp.float32(_K)),
                         axis=1, keepdims=True)
        codes_ref[:, lvl:lvl + 1] = code_f.astype(jnp.int32)
        onehot = (iota_f == code_f).astype(bf16)
        s = jnp.dot(onehot, hcat, preferred_element_type=f32)
        e = (s[:, 128:128 + _L] + s[:, 256:256 + _L]) + s[:, :_L]
        quant = quant + r + (e - r)
        r = r - e
        loss_acc = loss_acc + jnp.sum(r * r)

    h = jnp.maximum(
        jnp.dot(quant.astype(bf16), dW1_ref[...].astype(bf16),
                preferred_element_type=f32)
        + db1_ref[...], 0.0)
    h = jnp.maximum(
        jnp.dot(h.astype(bf16), dW2_ref[...].astype(bf16),
                preferred_element_type=f32)
        + db2_ref[...], 0.0)
    recon_ref[...] = (
        jnp.dot(h.astype(bf16), dW3_ref[...].astype(bf16),
                preferred_element_type=f32)
        + db3_ref[...])

    loss_ref[...] = loss_acc.reshape(1, 1, 1)


@jax.jit
def kernel(x, eW1, eb1, eW2, eb2, eW3, eb3, dW1, db1, dW2, db2, dW3, db3,
           cb0, cb1, cb2):
    grid = _B // _BT
    rep = lambda i: (0, 0)

    c2all = jnp.stack([jnp.sum(cb * cb, axis=1) for cb in (cb0, cb1, cb2)])

    recon, codes, loss_parts = pl.pallas_call(
        _fused_body,
        grid=(grid,),
        in_specs=[
            pl.BlockSpec((_BT, _D), lambda i: (i, 0)),
            pl.BlockSpec((_D, _H), rep),
            pl.BlockSpec((1, _H), rep),
            pl.BlockSpec((_H, _H), rep),
            pl.BlockSpec((1, _H), rep),
            pl.BlockSpec((_H, _L), rep),
            pl.BlockSpec((1, _L), rep),
            pl.BlockSpec((_L, _H), rep),
            pl.BlockSpec((1, _H), rep),
            pl.BlockSpec((_H, _H), rep),
            pl.BlockSpec((1, _H), rep),
            pl.BlockSpec((_H, _D), rep),
            pl.BlockSpec((1, _D), rep),
            pl.BlockSpec((3, _K), rep),
        ] + [pl.BlockSpec((_K, _L), rep)] * 3,
        out_specs=[
            pl.BlockSpec((_BT, _D), lambda i: (i, 0)),
            pl.BlockSpec((_BT, 3), lambda i: (i, 0)),
            pl.BlockSpec((1, 1, 1), lambda i: (i, 0, 0)),
        ],
        out_shape=[
            jax.ShapeDtypeStruct((_B, _D), jnp.float32),
            jax.ShapeDtypeStruct((_B, 3), jnp.int32),
            jax.ShapeDtypeStruct((grid, 1, 1), jnp.float32),
        ],
        compiler_params=pltpu.CompilerParams(
            dimension_semantics=("parallel",)),
    )(x, eW1, eb1.reshape(1, -1), eW2, eb2.reshape(1, -1),
      eW3, eb3.reshape(1, -1), dW1, db1.reshape(1, -1),
      dW2, db2.reshape(1, -1), dW3, db3.reshape(1, -1),
      c2all, cb0, cb1, cb2)

    loss = jnp.sum(loss_parts) * ((1.0 + _BETA) / (_B * _L))
    return recon, codes, loss
